# baseline (speedup 1.0000x reference)
import jax
import jax.numpy as jnp
import numpy as np
from jax import lax
from jax.experimental import pallas as pl
from jax.experimental.pallas import tpu as pltpu
from jax.experimental.shard_map import shard_map

_NUM_ROUNDS = 5
_OUT_PAD = 128
_BLOCK_B = 16


def _make_body(block_b, S, D):
    n = block_b * S
    rows = n * S

    def body(x0_ref, a_ref, w1_ref, b1_ref, w2_ref, b2_ref,
             ow1_ref, ob1_ref, ow2_ref, out_ref):
        x = x0_ref[...]
        w1 = w1_ref[...]
        w2 = w2_ref[...]
        ow1 = ow1_ref[...]
        ow2 = ow2_ref[...]

        b1f = jnp.broadcast_to(b1_ref[...], (n, D))
        b2f = jnp.broadcast_to(b2_ref[...], (rows, D))
        ob1f = jnp.broadcast_to(ob1_ref[...], (n, D))

        a = a_ref[...].astype(jnp.float32)
        a_e = jnp.broadcast_to(
            a.reshape(n, S, 1), (n, S, D))

        def round_body(_, x):
            xb = x.astype(jnp.bfloat16)
            pq = jnp.dot(xb, w1, preferred_element_type=jnp.float32)
            p = (pq[:, :D] + b1f).astype(jnp.bfloat16)
            q = pq[:, D:].astype(jnp.bfloat16)
            p_g = jnp.broadcast_to(
                p.reshape(block_b, 1, S, D),
                (block_b, S, S, D)).reshape(n, S, D)
            h = jnp.maximum(q[:, None, :] + p_g,
                            jnp.bfloat16(0.0))
            hb = h.reshape(rows, D)
            m = jnp.maximum(
                jnp.dot(hb, w2, preferred_element_type=jnp.float32) + b2f,
                0.0)
            mw = m * a_e.reshape(rows, D)
            msg = jnp.sum(mw.reshape(block_b, S, S, D), axis=1)
            return x + msg.reshape(n, D)

        x = lax.fori_loop(0, _NUM_ROUNDS, round_body, x, unroll=True)

        xb = x.astype(jnp.bfloat16)
        ho = jnp.maximum(
            jnp.dot(xb, ow1, preferred_element_type=jnp.float32) + ob1f, 0.0)
        out_ref[...] = jnp.dot(
            ho.astype(jnp.bfloat16), ow2,
            preferred_element_type=jnp.float32).astype(jnp.bfloat16)

    return body


def _forward(tok, A, w1, b1, w2, b2, ow1, ob1, ow2, embed, NC):
    B, S = tok.shape
    D = embed.shape[1]
    block_b = _BLOCK_B
    num_blocks = B // block_b
    n = block_b * S

    x0 = jnp.take(embed, tok.reshape(-1), axis=0).astype(jnp.float32)

    def wspec(r, c):
        return pl.BlockSpec((r, c), lambda b: (0, 0))

    y = pl.pallas_call(
        _make_body(block_b, S, D),
        out_shape=jax.ShapeDtypeStruct((B * S, _OUT_PAD), jnp.bfloat16),
        grid=(num_blocks,),
        in_specs=[
            pl.BlockSpec((n, D), lambda b: (b, 0)),
            pl.BlockSpec((block_b, S, S), lambda b: (b, 0, 0)),
            wspec(D, 2 * D),
            wspec(1, D),
            wspec(D, D),
            wspec(1, D),
            wspec(D, D),
            wspec(1, D),
            wspec(D, _OUT_PAD),
        ],
        out_specs=pl.BlockSpec((n, _OUT_PAD), lambda b: (b, 0)),
        compiler_params=pltpu.CompilerParams(
            dimension_semantics=("parallel",)),
    )(x0, A, w1, b1, w2, b2, ow1, ob1, ow2)

    x_full = y[:, :NC].astype(jnp.float32).reshape(B, S, NC)
    out = x_full[:, 0, :]
    return out, x_full


def kernel(tok, A, embed, msg_fc1_w, msg_fc1_b, msg_fc2_w, msg_fc2_b,
           out_fc1_w, out_fc1_b, out_fc2_w):
    B, S = tok.shape
    D = embed.shape[1]
    NC = out_fc2_w.shape[0]
    assert B % _BLOCK_B == 0

    w1 = jnp.concatenate(
        [msg_fc1_w[:, :D].T, msg_fc1_w[:, D:].T], axis=1).astype(jnp.bfloat16)
    w2 = msg_fc2_w.T.astype(jnp.bfloat16)
    ow1 = out_fc1_w.T.astype(jnp.bfloat16)
    ow2 = jnp.zeros((D, _OUT_PAD), jnp.float32).at[:, :NC].set(
        out_fc2_w.T).astype(jnp.bfloat16)
    b1 = msg_fc1_b.reshape(1, D)
    b2 = msg_fc2_b.reshape(1, D)
    ob1 = out_fc1_b.reshape(1, D)

    A = A.astype(jnp.bfloat16)

    def run(tok_s, A_s, w1_r, b1_r, w2_r, b2_r, ow1_r, ob1_r, ow2_r, emb_r):
        return _forward(tok_s, A_s, w1_r, b1_r, w2_r, b2_r, ow1_r, ob1_r,
                        ow2_r, emb_r, NC)

    devs = jax.devices()
    if (len(devs) >= 2 and devs[0].platform == "tpu"
            and B % (2 * _BLOCK_B) == 0):
        mesh = jax.sharding.Mesh(np.array(devs[:2]), ("d",))
        P = jax.sharding.PartitionSpec
        rep = (P(),) * 8
        return shard_map(
            run, mesh=mesh,
            in_specs=(P("d"), P("d")) + rep,
            out_specs=(P("d"), P("d")),
            check_rep=False,
        )(tok, A, w1, b1, w2, b2, ow1, ob1, ow2, embed)
    return run(tok, A, w1, b1, w2, b2, ow1, ob1, ow2, embed)


# speedup vs baseline: 1.0500x; 1.0500x over previous
import jax
import jax.numpy as jnp
import numpy as np
from jax import lax
from jax.experimental import pallas as pl
from jax.experimental.pallas import tpu as pltpu
from jax.experimental.shard_map import shard_map

_NUM_ROUNDS = 5
_OUT_PAD = 128
_BLOCK_B = 16


def _make_body(block_b, S, D):
    n = block_b * S
    rows = n * S

    def body(x0_ref, a_ref, w1_ref, b1_ref, w2_ref, b2_ref,
             ow1_ref, ob1_ref, ow2_ref, out_ref):
        x = x0_ref[...]
        w1 = w1_ref[...]
        w2 = w2_ref[...]
        ow1 = ow1_ref[...]
        ow2 = ow2_ref[...]

        b1f = jnp.broadcast_to(b1_ref[...], (n, D))
        b2f = jnp.broadcast_to(b2_ref[...], (rows, D))
        ob1f = jnp.broadcast_to(ob1_ref[...], (n, D))

        a = a_ref[...]
        a_e = jnp.broadcast_to(
            a.reshape(n, S, 1), (n, S, D))

        def round_body(_, x):
            xb = x.astype(jnp.bfloat16)
            pq = jnp.dot(xb, w1, preferred_element_type=jnp.float32)
            p = (pq[:, :D] + b1f).astype(jnp.bfloat16)
            q = pq[:, D:].astype(jnp.bfloat16)
            p_g = jnp.broadcast_to(
                p.reshape(block_b, 1, S, D),
                (block_b, S, S, D)).reshape(n, S, D)
            h = jnp.maximum(q[:, None, :] + p_g,
                            jnp.bfloat16(0.0))
            hb = h.reshape(rows, D)
            m = jnp.maximum(
                jnp.dot(hb, w2, preferred_element_type=jnp.float32) + b2f,
                0.0)
            mw = m * a_e.reshape(rows, D)
            msg = jnp.sum(mw.reshape(block_b, S, S, D), axis=1)
            return x + msg.reshape(n, D)

        x = lax.fori_loop(0, _NUM_ROUNDS, round_body, x, unroll=True)

        xb = x.astype(jnp.bfloat16)
        ho = jnp.maximum(
            jnp.dot(xb, ow1, preferred_element_type=jnp.float32) + ob1f, 0.0)
        out_ref[...] = jnp.dot(ho.astype(jnp.bfloat16), ow2,
                               preferred_element_type=jnp.float32)

    return body


def _forward(tok, A, w1, b1, w2, b2, ow1, ob1, ow2, embed, NC):
    B, S = tok.shape
    D = embed.shape[1]
    block_b = _BLOCK_B
    num_blocks = B // block_b
    n = block_b * S

    x0 = jnp.take(embed, tok.reshape(-1), axis=0).astype(jnp.float32)

    def wspec(r, c):
        return pl.BlockSpec((r, c), lambda b: (0, 0))

    y = pl.pallas_call(
        _make_body(block_b, S, D),
        out_shape=jax.ShapeDtypeStruct((B * S, _OUT_PAD), jnp.float32),
        grid=(num_blocks,),
        in_specs=[
            pl.BlockSpec((n, D), lambda b: (b, 0)),
            pl.BlockSpec((block_b, S, S), lambda b: (b, 0, 0)),
            wspec(D, 2 * D),
            wspec(1, D),
            wspec(D, D),
            wspec(1, D),
            wspec(D, D),
            wspec(1, D),
            wspec(D, _OUT_PAD),
        ],
        out_specs=pl.BlockSpec((n, _OUT_PAD), lambda b: (b, 0)),
        compiler_params=pltpu.CompilerParams(
            dimension_semantics=("parallel",)),
    )(x0, A, w1, b1, w2, b2, ow1, ob1, ow2)

    x_full = y[:, :NC].reshape(B, S, NC)
    out = x_full[:, 0, :]
    return out, x_full


def kernel(tok, A, embed, msg_fc1_w, msg_fc1_b, msg_fc2_w, msg_fc2_b,
           out_fc1_w, out_fc1_b, out_fc2_w):
    B, S = tok.shape
    D = embed.shape[1]
    NC = out_fc2_w.shape[0]
    assert B % _BLOCK_B == 0

    w1 = jnp.concatenate(
        [msg_fc1_w[:, :D].T, msg_fc1_w[:, D:].T], axis=1).astype(jnp.bfloat16)
    w2 = msg_fc2_w.T.astype(jnp.bfloat16)
    ow1 = out_fc1_w.T.astype(jnp.bfloat16)
    ow2 = jnp.zeros((D, _OUT_PAD), jnp.float32).at[:, :NC].set(
        out_fc2_w.T).astype(jnp.bfloat16)
    b1 = msg_fc1_b.reshape(1, D)
    b2 = msg_fc2_b.reshape(1, D)
    ob1 = out_fc1_b.reshape(1, D)

    def run(tok_s, A_s, w1_r, b1_r, w2_r, b2_r, ow1_r, ob1_r, ow2_r, emb_r):
        return _forward(tok_s, A_s, w1_r, b1_r, w2_r, b2_r, ow1_r, ob1_r,
                        ow2_r, emb_r, NC)

    devs = jax.devices()
    if (len(devs) >= 2 and devs[0].platform == "tpu"
            and B % (2 * _BLOCK_B) == 0):
        mesh = jax.sharding.Mesh(np.array(devs[:2]), ("d",))
        P = jax.sharding.PartitionSpec
        rep = (P(),) * 8
        return shard_map(
            run, mesh=mesh,
            in_specs=(P("d"), P("d")) + rep,
            out_specs=(P("d"), P("d")),
            check_rep=False,
        )(tok, A, w1, b1, w2, b2, ow1, ob1, ow2, embed)
    return run(tok, A, w1, b1, w2, b2, ow1, ob1, ow2, embed)


# speedup vs baseline: 1.0555x; 1.0052x over previous
import jax
import jax.numpy as jnp
import numpy as np
from jax import lax
from jax.experimental import pallas as pl
from jax.experimental.pallas import tpu as pltpu
from jax.experimental.shard_map import shard_map

_NUM_ROUNDS = 5
_OUT_PAD = 128
_BLOCK_B = 24


def _make_body(block_b, S, D):
    n = block_b * S
    rows = n * S

    def body(x0_ref, a_ref, w1_ref, b1_ref, w2_ref, b2_ref,
             ow1_ref, ob1_ref, ow2_ref, out_ref):
        x = x0_ref[...]
        w1 = w1_ref[...]
        w2 = w2_ref[...]
        ow1 = ow1_ref[...]
        ow2 = ow2_ref[...]

        b1f = jnp.broadcast_to(b1_ref[...], (n, D))
        b2f = jnp.broadcast_to(b2_ref[...], (rows, D))
        ob1f = jnp.broadcast_to(ob1_ref[...], (n, D))

        a = a_ref[...]
        a_e = jnp.broadcast_to(
            a.reshape(n, S, 1), (n, S, D))

        def round_body(_, x):
            xb = x.astype(jnp.bfloat16)
            pq = jnp.dot(xb, w1, preferred_element_type=jnp.float32)
            p = (pq[:, :D] + b1f).astype(jnp.bfloat16)
            q = pq[:, D:].astype(jnp.bfloat16)
            p_g = jnp.broadcast_to(
                p.reshape(block_b, 1, S, D),
                (block_b, S, S, D)).reshape(n, S, D)
            h = jnp.maximum(q[:, None, :] + p_g,
                            jnp.bfloat16(0.0))
            hb = h.reshape(rows, D)
            m = jnp.maximum(
                jnp.dot(hb, w2, preferred_element_type=jnp.float32) + b2f,
                0.0)
            mw = m * a_e.reshape(rows, D)
            msg = jnp.sum(mw.reshape(block_b, S, S, D), axis=1)
            return x + msg.reshape(n, D)

        x = lax.fori_loop(0, _NUM_ROUNDS, round_body, x, unroll=True)

        xb = x.astype(jnp.bfloat16)
        ho = jnp.maximum(
            jnp.dot(xb, ow1, preferred_element_type=jnp.float32) + ob1f, 0.0)
        out_ref[...] = jnp.dot(ho.astype(jnp.bfloat16), ow2,
                               preferred_element_type=jnp.float32)

    return body


def _forward(tok, A, w1, b1, w2, b2, ow1, ob1, ow2, embed, NC):
    B, S = tok.shape
    D = embed.shape[1]
    block_b = _BLOCK_B
    num_blocks = B // block_b
    n = block_b * S

    x0 = jnp.take(embed, tok.reshape(-1), axis=0).astype(jnp.float32)

    def wspec(r, c):
        return pl.BlockSpec((r, c), lambda b: (0, 0))

    y = pl.pallas_call(
        _make_body(block_b, S, D),
        out_shape=jax.ShapeDtypeStruct((B * S, _OUT_PAD), jnp.float32),
        grid=(num_blocks,),
        in_specs=[
            pl.BlockSpec((n, D), lambda b: (b, 0)),
            pl.BlockSpec((block_b, S, S), lambda b: (b, 0, 0)),
            wspec(D, 2 * D),
            wspec(1, D),
            wspec(D, D),
            wspec(1, D),
            wspec(D, D),
            wspec(1, D),
            wspec(D, _OUT_PAD),
        ],
        out_specs=pl.BlockSpec((n, _OUT_PAD), lambda b: (b, 0)),
        compiler_params=pltpu.CompilerParams(
            dimension_semantics=("parallel",)),
    )(x0, A, w1, b1, w2, b2, ow1, ob1, ow2)

    x_full = y[:, :NC].reshape(B, S, NC)
    out = x_full[:, 0, :]
    return out, x_full


def kernel(tok, A, embed, msg_fc1_w, msg_fc1_b, msg_fc2_w, msg_fc2_b,
           out_fc1_w, out_fc1_b, out_fc2_w):
    B, S = tok.shape
    D = embed.shape[1]
    NC = out_fc2_w.shape[0]
    assert B % _BLOCK_B == 0

    w1 = jnp.concatenate(
        [msg_fc1_w[:, :D].T, msg_fc1_w[:, D:].T], axis=1).astype(jnp.bfloat16)
    w2 = msg_fc2_w.T.astype(jnp.bfloat16)
    ow1 = out_fc1_w.T.astype(jnp.bfloat16)
    ow2 = jnp.zeros((D, _OUT_PAD), jnp.float32).at[:, :NC].set(
        out_fc2_w.T).astype(jnp.bfloat16)
    b1 = msg_fc1_b.reshape(1, D)
    b2 = msg_fc2_b.reshape(1, D)
    ob1 = out_fc1_b.reshape(1, D)

    def run(tok_s, A_s, w1_r, b1_r, w2_r, b2_r, ow1_r, ob1_r, ow2_r, emb_r):
        return _forward(tok_s, A_s, w1_r, b1_r, w2_r, b2_r, ow1_r, ob1_r,
                        ow2_r, emb_r, NC)

    devs = jax.devices()
    if (len(devs) >= 2 and devs[0].platform == "tpu"
            and B % (2 * _BLOCK_B) == 0):
        mesh = jax.sharding.Mesh(np.array(devs[:2]), ("d",))
        P = jax.sharding.PartitionSpec
        rep = (P(),) * 8
        return shard_map(
            run, mesh=mesh,
            in_specs=(P("d"), P("d")) + rep,
            out_specs=(P("d"), P("d")),
            check_rep=False,
        )(tok, A, w1, b1, w2, b2, ow1, ob1, ow2, embed)
    return run(tok, A, w1, b1, w2, b2, ow1, ob1, ow2, embed)
